# baseline (speedup 1.0000x reference)
import jax
import jax.numpy as jnp
from jax import lax
from jax.experimental import pallas as pl
from jax.experimental.pallas import tpu as pltpu
from jax.experimental.pallas import tpu_sc as plsc

B = 16384
D = 64
L = 16
NC = 2
NS = 16
NW = NC * NS
B_PER_W = B // NW
CHUNK = 128
NCHUNK = B_PER_W // CHUNK
NBUF = 2


def _body(users, items, user_table2, item_table2, out,
          idx_u_v, idx_i_v, idx_uh, idx_ih, bufs_u, bufs_i, out_v,
          sem_u, sem_i):
  wid = lax.axis_index("s") * NC + lax.axis_index("c")
  base = wid * B_PER_W

  pltpu.sync_copy(users.at[pl.ds(base, B_PER_W)], idx_u_v)
  pltpu.sync_copy(items.at[pl.ds(base, B_PER_W)], idx_i_v)

  for c in range(NCHUNK):
    for g in range(CHUNK // L):
      sl_src = pl.ds(c * CHUNK + g * L, L)
      sl_dst = pl.ds(g * L, L)
      idx_uh[c, sl_dst] = idx_u_v[sl_src] >> 1
      idx_ih[c, sl_dst] = idx_i_v[sl_src] >> 1

  def fire(c):
    b = c % NBUF
    cp_u = pltpu.async_copy(user_table2.at[idx_uh.at[c]], bufs_u.at[b], sem_u)
    cp_i = pltpu.async_copy(item_table2.at[idx_ih.at[c]], bufs_i.at[b], sem_i)
    return cp_u, cp_i

  riota = lax.iota(jnp.int32, L)
  copies = {}
  for c in range(min(NBUF, NCHUNK)):
    copies[c] = fire(c)

  for c in range(NCHUNK):
    cp_u, cp_i = copies.pop(c)
    cp_u.wait()
    cp_i.wait()
    b = c % NBUF
    buf_u = bufs_u.at[b]
    buf_i = bufs_i.at[b]

    def block(j, carry, c=c, buf_u=buf_u, buf_i=buf_i):
      ro = j * L
      row_ids = riota + ro
      coff_u = (idx_u_v[pl.ds(c * CHUNK + ro, L)] & 1) * D
      coff_i = (idx_i_v[pl.ds(c * CHUNK + ro, L)] & 1) * D
      acc = jnp.zeros((L,), jnp.float32)
      for k in range(D):
        uv = plsc.load_gather(buf_u, [row_ids, coff_u + k])
        iv = plsc.load_gather(buf_i, [row_ids, coff_i + k])
        acc = acc + uv * iv
      out_v[pl.ds(c * CHUNK + ro, L)] = acc
      return carry

    lax.fori_loop(0, CHUNK // L, block, 0)

    nxt = c + NBUF
    if nxt < NCHUNK:
      copies[nxt] = fire(nxt)

  pltpu.sync_copy(out_v, out.at[pl.ds(base, B_PER_W)])


@jax.jit
def kernel(users, items, user_table, item_table):
  user_table2 = user_table.reshape(-1, 2 * D)
  item_table2 = item_table.reshape(-1, 2 * D)
  mesh = plsc.VectorSubcoreMesh(core_axis_name="c", subcore_axis_name="s")
  k = pl.kernel(
      _body,
      out_type=jax.ShapeDtypeStruct((B,), jnp.float32),
      mesh=mesh,
      scratch_types=[
          pltpu.VMEM((B_PER_W,), jnp.int32),
          pltpu.VMEM((B_PER_W,), jnp.int32),
          pltpu.VMEM((NCHUNK, CHUNK), jnp.int32),
          pltpu.VMEM((NCHUNK, CHUNK), jnp.int32),
          pltpu.VMEM((NBUF, CHUNK, 2 * D), jnp.float32),
          pltpu.VMEM((NBUF, CHUNK, 2 * D), jnp.float32),
          pltpu.VMEM((B_PER_W,), jnp.float32),
          pltpu.SemaphoreType.DMA,
          pltpu.SemaphoreType.DMA,
      ],
      compiler_params=pltpu.CompilerParams(needs_layout_passes=False),
  )
  return k(users, items, user_table2, item_table2)


# speedup vs baseline: 1.5567x; 1.5567x over previous
import jax
import jax.numpy as jnp
from jax import lax
from jax.experimental import pallas as pl
from jax.experimental.pallas import tpu as pltpu
from jax.experimental.pallas import tpu_sc as plsc

B = 16384
D = 64
L = 16
NC = 2
NS = 16
NW = NC * NS
B_PER_W = B // NW
PASS_ROWS = B_PER_W // 2
NSEM = 4


def _body(users, items, user_table, item_table, out,
          idx_u_v, idx_i_v, rows_u, rows_i, out_v, *sems):
  wid = lax.axis_index("s") * NC + lax.axis_index("c")
  base = wid * B_PER_W

  pltpu.sync_copy(users.at[pl.ds(base, B_PER_W)], idx_u_v)
  pltpu.sync_copy(items.at[pl.ds(base, B_PER_W)], idx_i_v)

  riota = lax.iota(jnp.int32, L)

  for p in range(2):
    poff = p * PASS_ROWS

    def issue(g, carry, poff=poff):
      uvec = idx_u_v[pl.ds(poff + g * L, L)]
      ivec = idx_i_v[pl.ds(poff + g * L, L)]
      for l in range(L):
        pltpu.async_copy(user_table.at[uvec[l]], rows_u.at[g * L + l],
                         sems[l % NSEM])
        pltpu.async_copy(item_table.at[ivec[l]], rows_i.at[g * L + l],
                         sems[l % NSEM])
      return carry

    lax.fori_loop(0, PASS_ROWS // L, issue, 0)

    def drain(r, carry):
      for s in range(NSEM):
        pltpu.make_async_copy(user_table.at[0], rows_u.at[0], sems[s]).wait()
        pltpu.make_async_copy(item_table.at[0], rows_i.at[0], sems[s]).wait()
      return carry

    lax.fori_loop(0, PASS_ROWS // NSEM, drain, 0)

    def block(j, carry):
      ro = j * L
      row_ids = riota + ro
      acc = jnp.zeros((L,), jnp.float32)
      for k in range(D):
        col = jnp.full((L,), k, jnp.int32)
        uv = plsc.load_gather(rows_u, [row_ids, col])
        iv = plsc.load_gather(rows_i, [row_ids, col])
        acc = acc + uv * iv
      out_v[pl.ds(ro, L)] = acc
      return carry

    lax.fori_loop(0, PASS_ROWS // L, block, 0)

    pltpu.sync_copy(out_v, out.at[pl.ds(base + poff, PASS_ROWS)])


@jax.jit
def kernel(users, items, user_table, item_table):
  mesh = plsc.VectorSubcoreMesh(core_axis_name="c", subcore_axis_name="s")
  k = pl.kernel(
      _body,
      out_type=jax.ShapeDtypeStruct((B,), jnp.float32),
      mesh=mesh,
      scratch_types=[
          pltpu.VMEM((B_PER_W,), jnp.int32),
          pltpu.VMEM((B_PER_W,), jnp.int32),
          pltpu.VMEM((PASS_ROWS, D), jnp.float32),
          pltpu.VMEM((PASS_ROWS, D), jnp.float32),
          pltpu.VMEM((PASS_ROWS,), jnp.float32),
      ] + [pltpu.SemaphoreType.DMA] * NSEM,
      compiler_params=pltpu.CompilerParams(needs_layout_passes=False),
  )
  return k(users, items, user_table, item_table)
